# baseline (speedup 1.0000x reference)
import functools

import jax
import jax.numpy as jnp
from jax import lax
from jax.experimental import pallas as pl
from jax.experimental.pallas import tpu as pltpu
from jax.experimental.pallas import tpu_sc as plsc

VOCAB = 1000000
EMBED = 32
BATCH = 16384

_info = plsc.get_sparse_core_info()
_NC, _NS, _L = _info.num_cores, _info.num_subcores, _info.num_lanes
_NW = _NC * _NS
_BPW = BATCH // _NW
_CHUNK = 128
_LIST = _CHUNK * EMBED


def _sc_kernel(tbl_hbm, idx_t_hbm, idx_c_hbm, w_hbm, b_hbm, out_hbm,
               idx_t_v, idx_c_v, lst_t_v, lst_c_v, g_t_v, g_c_v, out_v,
               w_v, b_v, sem_t, sem_c):
    wid = lax.axis_index("s") * _NC + lax.axis_index("c")
    base = wid * _BPW
    pltpu.sync_copy(idx_t_hbm.at[pl.ds(base, _BPW)], idx_t_v)
    pltpu.sync_copy(idx_c_hbm.at[pl.ds(base, _BPW)], idx_c_v)
    pltpu.sync_copy(w_hbm, w_v)
    pltpu.sync_copy(b_hbm, b_v)

    wv = w_v[...]
    bv = b_v[...]

    def build_lists(j, cbase):
        vt = idx_t_v[pl.ds(cbase + j * 16, 16)]
        vc = idx_c_v[pl.ds(cbase + j * 16, 16)]
        for e in range(EMBED):
            lst_t_v[pl.ds(e * _CHUNK + j * 16, 16)] = vt + jnp.int32(e * VOCAB)
            lst_c_v[pl.ds(e * _CHUNK + j * 16, 16)] = vc + jnp.int32(e * VOCAB)

    def compute_group(j, cbase):
        acc = jnp.zeros((16,), jnp.float32)
        for e in range(EMBED):
            tv = g_t_v[pl.ds(e * _CHUNK + j * 16, 16)]
            cv = g_c_v[pl.ds(e * _CHUNK + j * 16, 16)]
            acc = acc + tv * cv
        z = acc * wv + bv
        out_v[pl.ds(cbase + j * 16, 16)] = 1.0 / (1.0 + jnp.exp(-z))

    def chunk_body(chunk, carry):
        cbase = chunk * _CHUNK

        def lb(j, c):
            build_lists(j, cbase)
            return c

        lax.fori_loop(0, _CHUNK // 16, lb, 0)
        cp_t = pltpu.async_copy(tbl_hbm.at[lst_t_v], g_t_v, sem_t)
        cp_c = pltpu.async_copy(tbl_hbm.at[lst_c_v], g_c_v, sem_c)
        cp_t.wait()
        cp_c.wait()

        def cg(j, c):
            compute_group(j, cbase)
            return c

        lax.fori_loop(0, _CHUNK // 16, cg, 0)
        return carry

    lax.fori_loop(0, _BPW // _CHUNK, chunk_body, 0)
    pltpu.sync_copy(out_v, out_hbm.at[pl.ds(base, _BPW)])


@functools.partial(
    pl.kernel,
    out_type=jax.ShapeDtypeStruct((BATCH,), jnp.float32),
    mesh=plsc.VectorSubcoreMesh(core_axis_name="c", subcore_axis_name="s"),
    compiler_params=pltpu.CompilerParams(
        needs_layout_passes=False, use_tc_tiling_on_sc=False),
    scratch_types=[
        pltpu.VMEM((_BPW,), jnp.int32),
        pltpu.VMEM((_BPW,), jnp.int32),
        pltpu.VMEM((_LIST,), jnp.int32),
        pltpu.VMEM((_LIST,), jnp.int32),
        pltpu.VMEM((_LIST,), jnp.float32),
        pltpu.VMEM((_LIST,), jnp.float32),
        pltpu.VMEM((_BPW,), jnp.float32),
        pltpu.VMEM((16,), jnp.float32),
        pltpu.VMEM((16,), jnp.float32),
        pltpu.SemaphoreType.DMA,
        pltpu.SemaphoreType.DMA,
    ],
)
def _context_model_sc(tbl_hbm, idx_t_hbm, idx_c_hbm, w_hbm, b_hbm, out_hbm,
                      idx_t_v, idx_c_v, lst_t_v, lst_c_v, g_t_v, g_c_v,
                      out_v, w_v, b_v, sem_t, sem_c):
    _sc_kernel(tbl_hbm, idx_t_hbm, idx_c_hbm, w_hbm, b_hbm, out_hbm,
               idx_t_v, idx_c_v, lst_t_v, lst_c_v, g_t_v, g_c_v, out_v,
               w_v, b_v, sem_t, sem_c)


def kernel(inputs, table, W, b):
    idx_t = inputs[:, 0].astype(jnp.int32)
    idx_c = inputs[:, 1].astype(jnp.int32)
    tbl_flat = table.T.reshape(VOCAB * EMBED)
    w16 = jnp.full((16,), W[0, 0], dtype=jnp.float32)
    b16 = jnp.full((16,), b[0], dtype=jnp.float32)
    out = _context_model_sc(tbl_flat, idx_t, idx_c, w16, b16)
    return out.reshape(BATCH, 1)


# speedup vs baseline: 12.7937x; 12.7937x over previous
import functools

import jax
import jax.numpy as jnp
from jax import lax
from jax.experimental import pallas as pl
from jax.experimental.pallas import tpu as pltpu
from jax.experimental.pallas import tpu_sc as plsc

VOCAB = 1000000
EMBED = 32
BATCH = 16384

_VB = 7813
_ROWS = 4 * _VB * 8
_PAIRS = 3907
_TOT_IT = 4 * _PAIRS

_info = plsc.get_sparse_core_info()
_NC, _NS, _L = _info.num_cores, _info.num_subcores, _info.num_lanes
_NW = _NC * _NS
_BPW = BATCH // _NW
_CHUNK = 128
_LIST = _CHUNK * EMBED
_SLOTS = 8


def _iter_params(it):
    g = it // _PAIRS
    p = it % _PAIRS
    vb0 = jnp.where(p == _PAIRS - 1, 2 * p - 1, 2 * p)
    return g, vb0


def _detile_kernel(tblT_hbm, out_hbm, buf_v, idx_v, sem_i, sem_o):
    wid = lax.axis_index("s") * _NC + lax.axis_index("c")
    n_i = jnp.int32((_TOT_IT + _NW - 1) // _NW)
    n_i = jnp.where(wid < jnp.int32(_TOT_IT % _NW), n_i, n_i - 1)
    n_chunks = (n_i + (_SLOTS - 1)) // _SLOTS

    def in_copy(b, it):
        g, vb0 = _iter_params(it)
        col = vb0 * 128
        c0 = pltpu.make_async_copy(
            tblT_hbm.at[pl.ds(g * 8, 8), pl.ds(col, 128)],
            buf_v.at[b, pl.ds(0, 8)], sem_i)
        c1 = pltpu.make_async_copy(
            tblT_hbm.at[pl.ds(g * 8, 8), pl.ds(col + 128, 128)],
            buf_v.at[b, pl.ds(8, 8)], sem_i)
        return c0, c1

    def out_copy(b, it):
        g, vb0 = _iter_params(it)
        return pltpu.make_async_copy(buf_v.at[b], out_hbm.at[idx_v.at[b]],
                                     sem_o)

    def chunk_body(ci, carry):
        def slot_it(b):
            return (ci * _SLOTS + b) * _NW + wid

        def guard(b, fn):
            i_local = ci * _SLOTS + b

            @pl.when(i_local < n_i)
            def _():
                fn()

        for b in range(_SLOTS):
            def fire_in(b=b):
                c0, c1 = in_copy(b, slot_it(b))
                c0.start()
                c1.start()
            guard(b, fire_in)
        for b in range(_SLOTS):
            def drain_in(b=b):
                c0, c1 = in_copy(b, slot_it(b))
                c0.wait()
                c1.wait()
            guard(b, drain_in)
        for b in range(_SLOTS):
            def fire_out(b=b):
                it = slot_it(b)
                g, vb0 = _iter_params(it)
                base = (g * _VB + vb0) * 8
                idx_v[b, :] = base + lax.iota(jnp.int32, 16)
                out_copy(b, it).start()
            guard(b, fire_out)
        for b in range(_SLOTS):
            def drain_out(b=b):
                out_copy(b, slot_it(b)).wait()
            guard(b, drain_out)
        return carry

    lax.fori_loop(0, n_chunks, chunk_body, 0)


@functools.partial(
    pl.kernel,
    out_type=jax.ShapeDtypeStruct((_ROWS, 128), jnp.float32),
    mesh=plsc.VectorSubcoreMesh(core_axis_name="c", subcore_axis_name="s"),
    compiler_params=pltpu.CompilerParams(
        needs_layout_passes=False, use_tc_tiling_on_sc=True),
    scratch_types=[
        pltpu.VMEM((_SLOTS, 16, 128), jnp.float32),
        pltpu.VMEM((_SLOTS, 16), jnp.int32),
        pltpu.SemaphoreType.DMA,
        pltpu.SemaphoreType.DMA,
    ],
)
def _detile_sc(tblT_hbm, out_hbm, buf_v, idx_v, sem_i, sem_o):
    _detile_kernel(tblT_hbm, out_hbm, buf_v, idx_v, sem_i, sem_o)


def _gather_kernel(tbl_hbm, idx_t_hbm, idx_c_hbm, w_hbm, b_hbm, out_hbm,
                   idx_t_v, idx_c_v, lst_t_v, lst_c_v, g_t_v, g_c_v, out_v,
                   w_v, b_v, sem_t, sem_c):
    wid = lax.axis_index("s") * _NC + lax.axis_index("c")
    base = wid * _BPW
    pltpu.sync_copy(idx_t_hbm.at[pl.ds(base, _BPW)], idx_t_v)
    pltpu.sync_copy(idx_c_hbm.at[pl.ds(base, _BPW)], idx_c_v)
    pltpu.sync_copy(w_hbm, w_v)
    pltpu.sync_copy(b_hbm, b_v)

    wv = w_v[...]
    bv = b_v[...]

    def build_lists(j, cbase):
        vt = idx_t_v[pl.ds(cbase + j * 16, 16)]
        vc = idx_c_v[pl.ds(cbase + j * 16, 16)]
        ht = lax.shift_left(lax.shift_right_logical(vt, 7), 10) \
            + jnp.bitwise_and(vt, 127)
        hc = lax.shift_left(lax.shift_right_logical(vc, 7), 10) \
            + jnp.bitwise_and(vc, 127)
        for e in range(EMBED):
            ce = jnp.int32(((e >> 3) * _VB) * 1024 + (e & 7) * 128)
            lst_t_v[pl.ds(e * _CHUNK + j * 16, 16)] = ht + ce
            lst_c_v[pl.ds(e * _CHUNK + j * 16, 16)] = hc + ce

    def compute_group(j, cbase):
        acc = jnp.zeros((16,), jnp.float32)
        for e in range(EMBED):
            tv = g_t_v[pl.ds(e * _CHUNK + j * 16, 16)]
            cv = g_c_v[pl.ds(e * _CHUNK + j * 16, 16)]
            acc = acc + tv * cv
        z = acc * wv + bv
        out_v[pl.ds(cbase + j * 16, 16)] = 1.0 / (1.0 + jnp.exp(-z))

    def chunk_body(chunk, carry):
        cbase = chunk * _CHUNK

        def lb(j, c):
            build_lists(j, cbase)
            return c

        lax.fori_loop(0, _CHUNK // 16, lb, 0)
        cp_t = pltpu.async_copy(tbl_hbm.at[lst_t_v], g_t_v, sem_t)
        cp_c = pltpu.async_copy(tbl_hbm.at[lst_c_v], g_c_v, sem_c)
        cp_t.wait()
        cp_c.wait()

        def cg(j, c):
            compute_group(j, cbase)
            return c

        lax.fori_loop(0, _CHUNK // 16, cg, 0)
        return carry

    lax.fori_loop(0, _BPW // _CHUNK, chunk_body, 0)
    pltpu.sync_copy(out_v, out_hbm.at[pl.ds(base, _BPW)])


@functools.partial(
    pl.kernel,
    out_type=jax.ShapeDtypeStruct((BATCH,), jnp.float32),
    mesh=plsc.VectorSubcoreMesh(core_axis_name="c", subcore_axis_name="s"),
    compiler_params=pltpu.CompilerParams(
        needs_layout_passes=False, use_tc_tiling_on_sc=False),
    scratch_types=[
        pltpu.VMEM((_BPW,), jnp.int32),
        pltpu.VMEM((_BPW,), jnp.int32),
        pltpu.VMEM((_LIST,), jnp.int32),
        pltpu.VMEM((_LIST,), jnp.int32),
        pltpu.VMEM((_LIST,), jnp.float32),
        pltpu.VMEM((_LIST,), jnp.float32),
        pltpu.VMEM((_BPW,), jnp.float32),
        pltpu.VMEM((16,), jnp.float32),
        pltpu.VMEM((16,), jnp.float32),
        pltpu.SemaphoreType.DMA,
        pltpu.SemaphoreType.DMA,
    ],
)
def _context_model_sc(tbl_hbm, idx_t_hbm, idx_c_hbm, w_hbm, b_hbm, out_hbm,
                      idx_t_v, idx_c_v, lst_t_v, lst_c_v, g_t_v, g_c_v,
                      out_v, w_v, b_v, sem_t, sem_c):
    _gather_kernel(tbl_hbm, idx_t_hbm, idx_c_hbm, w_hbm, b_hbm, out_hbm,
                   idx_t_v, idx_c_v, lst_t_v, lst_c_v, g_t_v, g_c_v, out_v,
                   w_v, b_v, sem_t, sem_c)


def kernel(inputs, table, W, b):
    idx_t = inputs[:, 0].astype(jnp.int32)
    idx_c = inputs[:, 1].astype(jnp.int32)
    tbl_emaj = _detile_sc(table.T)
    tbl_flat = tbl_emaj.reshape(_ROWS * 128)
    w16 = jnp.full((16,), W[0, 0], dtype=jnp.float32)
    b16 = jnp.full((16,), b[0], dtype=jnp.float32)
    out = _context_model_sc(tbl_flat, idx_t, idx_c, w16, b16)
    return out.reshape(BATCH, 1)
